# Initial kernel scaffold; baseline (speedup 1.0000x reference)
import jax
import jax.numpy as jnp
from jax.experimental import pallas as pl


def kernel(hidden_states, weight):
    raise NotImplementedError("write your pallas kernel here")


# speedup vs baseline: 1.0799x; 1.0799x over previous
import functools

import jax
import jax.numpy as jnp
from jax.experimental import pallas as pl

D_MODEL = 768
N_EXPERTS = 64
TOP_K = 8
BLOCK = 1024


def _router_kernel(x_ref, w_ref, idx_ref, wgt_ref):
    x = x_ref[...]
    w = w_ref[...]
    logits = jax.lax.dot_general(
        x, w, (((1,), (1,)), ((), ())), preferred_element_type=jnp.float32
    )
    iota = jax.lax.broadcasted_iota(jnp.int32, logits.shape, 1)
    cur = logits
    vals, idxs = [], []
    for _ in range(TOP_K):
        m = jnp.max(cur, axis=1, keepdims=True)
        am = jnp.min(jnp.where(cur == m, iota, N_EXPERTS), axis=1, keepdims=True)
        vals.append(m)
        idxs.append(am)
        cur = jnp.where(iota == am, -jnp.inf, cur)
    v = jnp.concatenate(vals, axis=1)
    i = jnp.concatenate(idxs, axis=1)
    e = jnp.exp(v - v[:, :1])
    wgt = e / jnp.sum(e, axis=1, keepdims=True)
    idx_ref[...] = i
    wgt_ref[...] = wgt


@jax.jit
def kernel(hidden_states, weight):
    b, s, h = hidden_states.shape
    n = b * s
    hs = hidden_states.reshape(n, h)
    idx, wgt = pl.pallas_call(
        _router_kernel,
        grid=(n // BLOCK,),
        in_specs=[
            pl.BlockSpec((BLOCK, h), lambda i: (i, 0)),
            pl.BlockSpec((N_EXPERTS, h), lambda i: (0, 0)),
        ],
        out_specs=[
            pl.BlockSpec((BLOCK, TOP_K), lambda i: (i, 0)),
            pl.BlockSpec((BLOCK, TOP_K), lambda i: (i, 0)),
        ],
        out_shape=[
            jax.ShapeDtypeStruct((n, TOP_K), jnp.int32),
            jax.ShapeDtypeStruct((n, TOP_K), jnp.float32),
        ],
    )(hs, weight)
    return idx, wgt, jnp.zeros((), jnp.float32)


# speedup vs baseline: 2.0306x; 1.8803x over previous
import jax
import jax.numpy as jnp
from jax.experimental import pallas as pl

D_MODEL = 768
N_EXPERTS = 64
TOP_K = 8
BLOCK = 1024


def _router_kernel(x_ref, w_ref, idx_ref, wgt_ref):
    x = x_ref[...]
    w = w_ref[...]
    lt = jax.lax.dot_general(
        w, x, (((1,), (1,)), ((), ())), preferred_element_type=jnp.float32
    )
    iota = jax.lax.broadcasted_iota(jnp.int32, lt.shape, 0).astype(jnp.float32)
    cur = lt
    vals, idxs = [], []
    for _ in range(TOP_K):
        m = jnp.max(cur, axis=0, keepdims=True)
        am = jnp.min(
            jnp.where(cur == m, iota, jnp.float32(N_EXPERTS)),
            axis=0, keepdims=True,
        )
        vals.append(m)
        idxs.append(am)
        cur = jnp.where(iota == am, -jnp.inf, cur)
    v = jnp.concatenate(vals, axis=0)
    fi = jnp.concatenate(idxs, axis=0)
    e = jnp.exp(v - v[:1])
    wgt = e / jnp.sum(e, axis=0, keepdims=True)
    idx_ref[...] = fi.astype(jnp.int32).T
    wgt_ref[...] = wgt.T


@jax.jit
def kernel(hidden_states, weight):
    b, s, h = hidden_states.shape
    n = b * s
    hs = hidden_states.reshape(n, h)
    idx, wgt = pl.pallas_call(
        _router_kernel,
        grid=(n // BLOCK,),
        in_specs=[
            pl.BlockSpec((BLOCK, h), lambda i: (i, 0)),
            pl.BlockSpec((N_EXPERTS, h), lambda i: (0, 0)),
        ],
        out_specs=[
            pl.BlockSpec((BLOCK, TOP_K), lambda i: (i, 0)),
            pl.BlockSpec((BLOCK, TOP_K), lambda i: (i, 0)),
        ],
        out_shape=[
            jax.ShapeDtypeStruct((n, TOP_K), jnp.int32),
            jax.ShapeDtypeStruct((n, TOP_K), jnp.float32),
        ],
    )(hs, weight)
    return idx, wgt, jnp.zeros((), jnp.float32)


# speedup vs baseline: 3.1807x; 1.5663x over previous
import jax
import jax.numpy as jnp
from jax.experimental import pallas as pl

D_MODEL = 768
N_EXPERTS = 64
TOP_K = 8
BLOCK = 1024


def _router_kernel(x_ref, w_ref, idx_ref, wgt_ref):
    x = x_ref[...]
    w = w_ref[...]
    lt = jax.lax.dot_general(
        w, x, (((1,), (1,)), ((), ())), preferred_element_type=jnp.float32
    )
    iota = jax.lax.broadcasted_iota(jnp.int32, lt.shape, 0).astype(jnp.float32)
    cur = lt
    vals, idxs = [], []
    for _ in range(TOP_K):
        m = jnp.max(cur, axis=0, keepdims=True)
        am = jnp.min(
            jnp.where(cur == m, iota, jnp.float32(N_EXPERTS)),
            axis=0, keepdims=True,
        )
        vals.append(m)
        idxs.append(am)
        cur = jnp.where(iota == am, -jnp.inf, cur)
    v = jnp.concatenate(vals, axis=0)
    fi = jnp.concatenate(idxs, axis=0)
    e = jnp.exp(v - v[:1])
    wgt = e / jnp.sum(e, axis=0, keepdims=True)
    idx_ref[...] = fi.astype(jnp.int32)
    wgt_ref[...] = wgt


@jax.jit
def kernel(hidden_states, weight):
    b, s, h = hidden_states.shape
    n = b * s
    hs = hidden_states.reshape(n, h)
    idx, wgt = pl.pallas_call(
        _router_kernel,
        grid=(n // BLOCK,),
        in_specs=[
            pl.BlockSpec((BLOCK, h), lambda i: (i, 0)),
            pl.BlockSpec((N_EXPERTS, h), lambda i: (0, 0)),
        ],
        out_specs=[
            pl.BlockSpec((TOP_K, BLOCK), lambda i: (0, i)),
            pl.BlockSpec((TOP_K, BLOCK), lambda i: (0, i)),
        ],
        out_shape=[
            jax.ShapeDtypeStruct((TOP_K, n), jnp.int32),
            jax.ShapeDtypeStruct((TOP_K, n), jnp.float32),
        ],
    )(hs, weight)
    return idx.T, wgt.T, jnp.zeros((), jnp.float32)


# speedup vs baseline: 3.2016x; 1.0066x over previous
import jax
import jax.numpy as jnp
from jax.experimental import pallas as pl
from jax.experimental.pallas import tpu as pltpu

D_MODEL = 768
N_EXPERTS = 64
TOP_K = 8
BLOCK = 1024


def _router_kernel(x_ref, w_ref, idx_ref, wgt_ref):
    x = x_ref[...]
    w = w_ref[...]
    lt = jax.lax.dot_general(
        w, x, (((1,), (1,)), ((), ())), preferred_element_type=jnp.float32
    )
    iota = jax.lax.broadcasted_iota(jnp.int32, lt.shape, 0).astype(jnp.float32)
    cur = lt
    vals, idxs = [], []
    for _ in range(TOP_K):
        m = jnp.max(cur, axis=0, keepdims=True)
        am = jnp.min(
            jnp.where(cur == m, iota, jnp.float32(N_EXPERTS)),
            axis=0, keepdims=True,
        )
        vals.append(m)
        idxs.append(am)
        cur = jnp.where(iota == am, -jnp.inf, cur)
    v = jnp.concatenate(vals, axis=0)
    fi = jnp.concatenate(idxs, axis=0)
    e = jnp.exp(v - v[:1])
    wgt = e / jnp.sum(e, axis=0, keepdims=True)
    idx_ref[...] = fi.astype(jnp.int32)
    wgt_ref[...] = wgt


@jax.jit
def kernel(hidden_states, weight):
    b, s, h = hidden_states.shape
    n = b * s
    hs = hidden_states.reshape(n, h)
    idx, wgt = pl.pallas_call(
        _router_kernel,
        grid=(n // BLOCK,),
        in_specs=[
            pl.BlockSpec((BLOCK, h), lambda i: (i, 0)),
            pl.BlockSpec((N_EXPERTS, h), lambda i: (0, 0)),
        ],
        out_specs=[
            pl.BlockSpec((TOP_K, BLOCK), lambda i: (0, i)),
            pl.BlockSpec((TOP_K, BLOCK), lambda i: (0, i)),
        ],
        out_shape=[
            jax.ShapeDtypeStruct((TOP_K, n), jnp.int32),
            jax.ShapeDtypeStruct((TOP_K, n), jnp.float32),
        ],
        compiler_params=pltpu.CompilerParams(
            dimension_semantics=("parallel",),
        ),
    )(hs, weight)
    return idx.T, wgt.T, jnp.zeros((), jnp.float32)


# speedup vs baseline: 3.8185x; 1.1927x over previous
import jax
import jax.numpy as jnp
from jax.experimental import pallas as pl
from jax.experimental.pallas import tpu as pltpu

D_MODEL = 768
N_EXPERTS = 64
TOP_K = 8
BLOCK = 2048


def _router_kernel(x_ref, w_ref, idx_ref, wgt_ref):
    x = x_ref[...]
    w = w_ref[...]
    lt = jax.lax.dot_general(
        w, x, (((1,), (1,)), ((), ())), preferred_element_type=jnp.float32
    )
    iota = jax.lax.broadcasted_iota(jnp.int32, lt.shape, 0).astype(jnp.float32)
    cur = lt
    vals, idxs = [], []
    for _ in range(TOP_K):
        m = jnp.max(cur, axis=0, keepdims=True)
        am = jnp.min(
            jnp.where(cur == m, iota, jnp.float32(N_EXPERTS)),
            axis=0, keepdims=True,
        )
        vals.append(m)
        idxs.append(am)
        cur = jnp.where(iota == am, -jnp.inf, cur)
    v = jnp.concatenate(vals, axis=0)
    fi = jnp.concatenate(idxs, axis=0)
    e = jnp.exp(v - v[:1])
    wgt = e / jnp.sum(e, axis=0, keepdims=True)
    idx_ref[...] = fi.astype(jnp.int32)
    wgt_ref[...] = wgt


@jax.jit
def kernel(hidden_states, weight):
    b, s, h = hidden_states.shape
    n = b * s
    hs = hidden_states.reshape(n, h)
    idx, wgt = pl.pallas_call(
        _router_kernel,
        grid=(n // BLOCK,),
        in_specs=[
            pl.BlockSpec((BLOCK, h), lambda i: (i, 0)),
            pl.BlockSpec((N_EXPERTS, h), lambda i: (0, 0)),
        ],
        out_specs=[
            pl.BlockSpec((TOP_K, BLOCK), lambda i: (0, i)),
            pl.BlockSpec((TOP_K, BLOCK), lambda i: (0, i)),
        ],
        out_shape=[
            jax.ShapeDtypeStruct((TOP_K, n), jnp.int32),
            jax.ShapeDtypeStruct((TOP_K, n), jnp.float32),
        ],
        compiler_params=pltpu.CompilerParams(
            dimension_semantics=("parallel",),
        ),
    )(hs, weight)
    return idx.T, wgt.T, jnp.zeros((), jnp.float32)


# speedup vs baseline: 4.0523x; 1.0612x over previous
import jax
import jax.numpy as jnp
from jax.experimental import pallas as pl
from jax.experimental.pallas import tpu as pltpu

D_MODEL = 768
N_EXPERTS = 64
TOP_K = 8
BLOCK = 4096


def _router_kernel(x_ref, w_ref, idx_ref, wgt_ref):
    x = x_ref[...]
    w = w_ref[...]
    lt = jax.lax.dot_general(
        w, x, (((1,), (1,)), ((), ())), preferred_element_type=jnp.float32
    )
    iota = jax.lax.broadcasted_iota(jnp.int32, lt.shape, 0).astype(jnp.float32)
    cur = lt
    vals, idxs = [], []
    for _ in range(TOP_K):
        m = jnp.max(cur, axis=0, keepdims=True)
        am = jnp.min(
            jnp.where(cur == m, iota, jnp.float32(N_EXPERTS)),
            axis=0, keepdims=True,
        )
        vals.append(m)
        idxs.append(am)
        cur = jnp.where(iota == am, -jnp.inf, cur)
    v = jnp.concatenate(vals, axis=0)
    fi = jnp.concatenate(idxs, axis=0)
    e = jnp.exp(v - v[:1])
    wgt = e / jnp.sum(e, axis=0, keepdims=True)
    idx_ref[...] = fi.astype(jnp.int32)
    wgt_ref[...] = wgt


@jax.jit
def kernel(hidden_states, weight):
    b, s, h = hidden_states.shape
    n = b * s
    hs = hidden_states.reshape(n, h)
    idx, wgt = pl.pallas_call(
        _router_kernel,
        grid=(n // BLOCK,),
        in_specs=[
            pl.BlockSpec((BLOCK, h), lambda i: (i, 0)),
            pl.BlockSpec((N_EXPERTS, h), lambda i: (0, 0)),
        ],
        out_specs=[
            pl.BlockSpec((TOP_K, BLOCK), lambda i: (0, i)),
            pl.BlockSpec((TOP_K, BLOCK), lambda i: (0, i)),
        ],
        out_shape=[
            jax.ShapeDtypeStruct((TOP_K, n), jnp.int32),
            jax.ShapeDtypeStruct((TOP_K, n), jnp.float32),
        ],
        compiler_params=pltpu.CompilerParams(
            dimension_semantics=("parallel",),
        ),
    )(hs, weight)
    return idx.T, wgt.T, jnp.zeros((), jnp.float32)


# speedup vs baseline: 4.0677x; 1.0038x over previous
import jax
import jax.numpy as jnp
from jax.experimental import pallas as pl
from jax.experimental.pallas import tpu as pltpu

D_MODEL = 768
N_EXPERTS = 64
TOP_K = 8
CHUNK = 2048
NBUF = 4


def _router_kernel(x_hbm, w_ref, idx_ref, wgt_ref, xbuf, sems):
    i = pl.program_id(0)
    nc = pl.num_programs(0)

    def copy(j, slot):
        return pltpu.make_async_copy(
            x_hbm.at[pl.ds(j * CHUNK, CHUNK), :], xbuf.at[slot], sems.at[slot]
        )

    @pl.when(i == 0)
    def _():
        for j in range(NBUF):
            copy(j, j).start()

    @pl.when((i > 0) & (i + NBUF - 1 < nc))
    def _():
        j = i + NBUF - 1
        copy(j, jax.lax.rem(j, NBUF)).start()

    slot = jax.lax.rem(i, NBUF)
    copy(i, slot).wait()

    x = xbuf[slot]
    w = w_ref[...]
    lt = jax.lax.dot_general(
        w, x, (((1,), (1,)), ((), ())), preferred_element_type=jnp.float32
    )
    iota = jax.lax.broadcasted_iota(jnp.int32, lt.shape, 0).astype(jnp.float32)
    cur = lt
    vals, idxs = [], []
    for _ in range(TOP_K):
        m = jnp.max(cur, axis=0, keepdims=True)
        am = jnp.min(
            jnp.where(cur == m, iota, jnp.float32(N_EXPERTS)),
            axis=0, keepdims=True,
        )
        vals.append(m)
        idxs.append(am)
        cur = jnp.where(iota == am, -jnp.inf, cur)
    v = jnp.concatenate(vals, axis=0)
    fi = jnp.concatenate(idxs, axis=0)
    e = jnp.exp(v - v[:1])
    wgt = e / jnp.sum(e, axis=0, keepdims=True)
    idx_ref[...] = fi.astype(jnp.int32)
    wgt_ref[...] = wgt


@jax.jit
def kernel(hidden_states, weight):
    b, s, h = hidden_states.shape
    n = b * s
    hs = hidden_states.reshape(n, h)
    idx, wgt = pl.pallas_call(
        _router_kernel,
        grid=(n // CHUNK,),
        in_specs=[
            pl.BlockSpec(memory_space=pltpu.MemorySpace.HBM),
            pl.BlockSpec((N_EXPERTS, h), lambda i: (0, 0)),
        ],
        out_specs=[
            pl.BlockSpec((TOP_K, CHUNK), lambda i: (0, i)),
            pl.BlockSpec((TOP_K, CHUNK), lambda i: (0, i)),
        ],
        out_shape=[
            jax.ShapeDtypeStruct((TOP_K, n), jnp.int32),
            jax.ShapeDtypeStruct((TOP_K, n), jnp.float32),
        ],
        scratch_shapes=[
            pltpu.VMEM((NBUF, CHUNK, D_MODEL), jnp.float32),
            pltpu.SemaphoreType.DMA((NBUF,)),
        ],
    )(hs, weight)
    return idx.T, wgt.T, jnp.zeros((), jnp.float32)
